# baseline (speedup 1.0000x reference)
import functools

import jax
import jax.numpy as jnp
from jax.experimental import pallas as pl
from jax.experimental.pallas import tpu as pltpu

N = 4096
EB = 8192
G = 128
H = 128
FD = 28
L = 4
K = 32
CUTOFF = 10.0


def _head_body(np_ref, ep_ref, vm_ref, w1_ref, b1_ref, w2_ref, b2_ref,
               w3_ref, b3_ref, out_ref):
    np_blk = np_ref[...]
    ep_blk = ep_ref[...]
    h_pair = jnp.concatenate([np_blk, ep_blk], axis=-1)
    x1 = jax.nn.relu(h_pair @ w1_ref[...] + b1_ref[...])
    x2 = jax.nn.relu(x1 @ w2_ref[...] + b2_ref[...])
    y = x2 @ w3_ref[...] + b3_ref[...]
    out_ref[...] = y * vm_ref[...]


def kernel(atom_type, r_feat, p_feat, pos, bond_index, bond_type, batch, time_step, bond_emb, atom_emb, W_feat, Wmsg_r, bmsg_r, Wmsg_p, bmsg_p, Wmsg_g, bmsg_g, Wupd, bupd, W1, b1, W2, b2, W3, b3):
    n = atom_type.shape[0]
    d2 = jnp.sum((pos[:, None, :] - pos[None, :, :]) ** 2, axis=-1)
    same = batch[:, None] == batch[None, :]
    eye = jnp.eye(n, dtype=bool)
    valid = same & (~eye)
    neg = jnp.where(valid, -d2, -jnp.inf)
    vals, nbr = jax.lax.top_k(neg, K)
    src = jnp.repeat(jnp.arange(n), K)
    dst = nbr.reshape(-1)
    valid_e = (vals > -(CUTOFF ** 2)).reshape(-1)
    vm = valid_e.astype(jnp.float32)[:, None]
    bt_r = bond_type.astype(jnp.int32)
    bt_p = ((bond_type + 3) % 100).astype(jnp.int32)
    adj_r = jnp.zeros((n, n), jnp.int32).at[bond_index[0], bond_index[1]].set(bt_r)
    adj_p = jnp.zeros((n, n), jnp.int32).at[bond_index[0], bond_index[1]].set(bt_p)
    t_gr = adj_r[src, dst]
    t_gp = adj_p[src, dst]
    edge_attr_global_r = jnp.take(bond_emb, t_gr, axis=0)
    edge_attr_global_p = jnp.take(bond_emb, t_gp, axis=0)
    edge_attr_r = jnp.take(bond_emb, bt_r, axis=0)
    edge_attr_p = jnp.take(bond_emb, bt_p, axis=0)
    edge_length = jnp.sqrt(jnp.sum((pos[src] - pos[dst]) ** 2, axis=-1, keepdims=True) + 1e-12)
    a_emb = jnp.take(atom_emb, atom_type, axis=0)
    fr = r_feat @ W_feat
    fp = p_feat @ W_feat
    h = jnp.concatenate([a_emb + fr, fp - fr], axis=-1)
    s, d = bond_index[0], bond_index[1]
    for l in range(L):
        hi = h[s]
        hj = h[d]
        m_r = jax.nn.relu(jnp.concatenate([hi, hj, edge_attr_r], -1) @ Wmsg_r[l] + bmsg_r[l])
        agg_r = jax.ops.segment_sum(m_r, d, num_segments=n)
        m_p = jax.nn.relu(jnp.concatenate([hi, hj, edge_attr_p], -1) @ Wmsg_p[l] + bmsg_p[l])
        agg_p = jax.ops.segment_sum(m_p, d, num_segments=n)
        hgi = h[src]
        hgj = h[dst]
        m_g = jax.nn.relu(jnp.concatenate([hgi, hgj, edge_length], -1) @ Wmsg_g[l] + bmsg_g[l]) * vm
        agg_g = jax.ops.segment_sum(m_g, dst, num_segments=n)
        h = h + jax.nn.relu(jnp.concatenate([h, agg_r + agg_p + agg_g], -1) @ Wupd[l] + bupd[l])
    node_attr = h
    edge_pair = edge_attr_global_r * edge_attr_global_p
    node_pair = node_attr[src] * node_attr[dst]

    E = n * K
    BLK = 1024
    out = pl.pallas_call(
        _head_body,
        out_shape=jax.ShapeDtypeStruct((E, 1), jnp.float32),
        grid=(E // BLK,),
        in_specs=[
            pl.BlockSpec((BLK, H), lambda i: (i, 0)),
            pl.BlockSpec((BLK, H), lambda i: (i, 0)),
            pl.BlockSpec((BLK, 1), lambda i: (i, 0)),
            pl.BlockSpec((2 * H, H), lambda i: (0, 0)),
            pl.BlockSpec((H,), lambda i: (0,)),
            pl.BlockSpec((H, H // 2), lambda i: (0, 0)),
            pl.BlockSpec((H // 2,), lambda i: (0,)),
            pl.BlockSpec((H // 2, 1), lambda i: (0, 0)),
            pl.BlockSpec((1,), lambda i: (0,)),
        ],
        out_specs=pl.BlockSpec((BLK, 1), lambda i: (i, 0)),
    )(node_pair, edge_pair, vm, W1, b1, W2, b2, W3, b3)
    return out


# speedup vs baseline: 1.9847x; 1.9847x over previous
import functools

import jax
import jax.numpy as jnp
from jax.experimental import pallas as pl
from jax.experimental.pallas import tpu as pltpu

N = 4096
EB = 8192
G = 128
H = 128
FD = 28
L = 4
K = 32
CUTOFF = 10.0

RB = 128
NRB = N // RB
BB = 128
NBB = EB // BB


def _knn_body(posT_blk_ref, posT_all_ref, bat_blk_ref, bat_all_ref,
              nbr_ref, d2k_ref):
    i = pl.program_id(0)
    d2 = None
    for c in range(3):
        a = posT_blk_ref[c, :]
        b = posT_all_ref[c, :]
        diff = a[:, None] - b[None, :]
        sq = diff * diff
        d2 = sq if d2 is None else d2 + sq
    row_ids = i * RB + jax.lax.broadcasted_iota(jnp.int32, (RB, 1), 0)
    col_ids = jax.lax.broadcasted_iota(jnp.int32, (RB, N), 1)
    same = bat_blk_ref[0, :][:, None] == bat_all_ref[0, :][None, :]
    valid = same & (col_ids != row_ids)
    inf = jnp.float32(jnp.inf)
    cur = jnp.where(valid, d2, inf)
    big = jnp.int32(2**30)
    for k in range(K):
        m = jnp.min(cur, axis=1)
        cand = jnp.where(cur == m[:, None], col_ids, big)
        idx = jnp.min(cand, axis=1)
        nbr_ref[:, k] = idx
        d2k_ref[:, k] = m
        cur = jnp.where(col_ids == idx[:, None], inf, cur)


def _knn(posT, batT):
    return pl.pallas_call(
        _knn_body,
        out_shape=(jax.ShapeDtypeStruct((N, K), jnp.int32),
                   jax.ShapeDtypeStruct((N, K), jnp.float32)),
        grid=(NRB,),
        in_specs=[
            pl.BlockSpec((8, RB), lambda i: (0, i)),
            pl.BlockSpec((8, N), lambda i: (0, 0)),
            pl.BlockSpec((8, RB), lambda i: (0, i)),
            pl.BlockSpec((8, N), lambda i: (0, 0)),
        ],
        out_specs=(pl.BlockSpec((RB, K), lambda i: (i, 0)),
                   pl.BlockSpec((RB, K), lambda i: (i, 0))),
    )(posT, posT, batT, batT)


def _tg_body(bsrc_ref, bdst_ref, btr_ref, btp_ref, nhi_ref, nlo_ref,
             tgr_ref, tgp_ref):
    i = pl.program_id(0)

    @pl.when(i == 0)
    def _():
        tgr_ref[...] = jnp.zeros_like(tgr_ref)
        tgp_ref[...] = jnp.zeros_like(tgp_ref)

    src = bsrc_ref[0, :]
    dst = bdst_ref[0, :].astype(jnp.float32)
    col_ids = jax.lax.broadcasted_iota(jnp.int32, (BB, N), 1)
    R = (col_ids == src[:, None]).astype(jnp.float32)
    rows_hi = jnp.dot(R, nhi_ref[...])
    rows_lo = jnp.dot(R, nlo_ref[...])
    rows = rows_hi * 256.0 + rows_lo
    match = (rows == dst[:, None]).astype(jnp.float32)
    Br = match * btr_ref[0, :].astype(jnp.float32)[:, None]
    Bp = match * btp_ref[0, :].astype(jnp.float32)[:, None]
    dn = (((0,), (0,)), ((), ()))
    tgr_ref[...] += jax.lax.dot_general(R, Br, dn)
    tgp_ref[...] += jax.lax.dot_general(R, Bp, dn)


def _tg(bsrc, bdst, btr, btp, nhi, nlo):
    return pl.pallas_call(
        _tg_body,
        out_shape=(jax.ShapeDtypeStruct((N, K), jnp.float32),
                   jax.ShapeDtypeStruct((N, K), jnp.float32)),
        grid=(NBB,),
        in_specs=[
            pl.BlockSpec((1, BB), lambda i: (0, i)),
            pl.BlockSpec((1, BB), lambda i: (0, i)),
            pl.BlockSpec((1, BB), lambda i: (0, i)),
            pl.BlockSpec((1, BB), lambda i: (0, i)),
            pl.BlockSpec((N, K), lambda i: (0, 0)),
            pl.BlockSpec((N, K), lambda i: (0, 0)),
        ],
        out_specs=(pl.BlockSpec((N, K), lambda i: (0, 0)),
                   pl.BlockSpec((N, K), lambda i: (0, 0))),
    )(bsrc, bdst, btr, btp, nhi, nlo)


def kernel(atom_type, r_feat, p_feat, pos, bond_index, bond_type, batch, time_step, bond_emb, atom_emb, W_feat, Wmsg_r, bmsg_r, Wmsg_p, bmsg_p, Wmsg_g, bmsg_g, Wupd, bupd, W1, b1, W2, b2, W3, b3):
    n = N
    posT = jnp.concatenate([pos.T, jnp.zeros((5, n), jnp.float32)], axis=0)
    batT = jnp.broadcast_to(batch.astype(jnp.int32)[None, :], (8, n))
    nbr, d2k = _knn(posT, batT)

    vm2 = (d2k < CUTOFF * CUTOFF).astype(jnp.float32)
    vm = vm2.reshape(-1)[:, None]
    elen2 = jnp.where(vm2 > 0, jnp.sqrt(d2k + 1e-12), 0.0)
    edge_length = elen2.reshape(-1)[:, None]
    src = jnp.repeat(jnp.arange(n), K)
    dst = nbr.reshape(-1)

    bt_r = bond_type.astype(jnp.int32)
    bt_p = ((bond_type + 3) % 100).astype(jnp.int32)
    nhi = jnp.floor_divide(nbr, 256).astype(jnp.float32)
    nlo = jnp.mod(nbr, 256).astype(jnp.float32)
    tgr_f, tgp_f = _tg(bond_index[0][None, :].astype(jnp.int32),
                       bond_index[1][None, :].astype(jnp.int32),
                       bt_r[None, :], bt_p[None, :], nhi, nlo)
    t_gr = tgr_f.reshape(-1).astype(jnp.int32)
    t_gp = tgp_f.reshape(-1).astype(jnp.int32)
    t_gr = jnp.clip(t_gr, 0, 99)
    t_gp = jnp.clip(t_gp, 0, 99)

    edge_attr_global_r = jnp.take(bond_emb, t_gr, axis=0)
    edge_attr_global_p = jnp.take(bond_emb, t_gp, axis=0)
    edge_attr_r = jnp.take(bond_emb, bt_r, axis=0)
    edge_attr_p = jnp.take(bond_emb, bt_p, axis=0)

    a_emb = jnp.take(atom_emb, atom_type, axis=0)
    fr = r_feat @ W_feat
    fp = p_feat @ W_feat
    h = jnp.concatenate([a_emb + fr, fp - fr], axis=-1)
    s, d = bond_index[0], bond_index[1]
    for l in range(L):
        hi = h[s]
        hj = h[d]
        m_r = jax.nn.relu(jnp.concatenate([hi, hj, edge_attr_r], -1) @ Wmsg_r[l] + bmsg_r[l])
        agg_r = jax.ops.segment_sum(m_r, d, num_segments=n)
        m_p = jax.nn.relu(jnp.concatenate([hi, hj, edge_attr_p], -1) @ Wmsg_p[l] + bmsg_p[l])
        agg_p = jax.ops.segment_sum(m_p, d, num_segments=n)
        hgi = h[src]
        hgj = h[dst]
        m_g = jax.nn.relu(jnp.concatenate([hgi, hgj, edge_length], -1) @ Wmsg_g[l] + bmsg_g[l]) * vm
        agg_g = jax.ops.segment_sum(m_g, dst, num_segments=n)
        h = h + jax.nn.relu(jnp.concatenate([h, agg_r + agg_p + agg_g], -1) @ Wupd[l] + bupd[l])

    node_attr = h
    edge_pair = edge_attr_global_r * edge_attr_global_p
    node_pair = node_attr[src] * node_attr[dst]

    E = n * K
    BLK = 1024
    out = pl.pallas_call(
        _head_body,
        out_shape=jax.ShapeDtypeStruct((E, 1), jnp.float32),
        grid=(E // BLK,),
        in_specs=[
            pl.BlockSpec((BLK, H), lambda i: (i, 0)),
            pl.BlockSpec((BLK, H), lambda i: (i, 0)),
            pl.BlockSpec((BLK, 1), lambda i: (i, 0)),
            pl.BlockSpec((2 * H, H), lambda i: (0, 0)),
            pl.BlockSpec((H,), lambda i: (0,)),
            pl.BlockSpec((H, H // 2), lambda i: (0, 0)),
            pl.BlockSpec((H // 2,), lambda i: (0,)),
            pl.BlockSpec((H // 2, 1), lambda i: (0, 0)),
            pl.BlockSpec((1,), lambda i: (0,)),
        ],
        out_specs=pl.BlockSpec((BLK, 1), lambda i: (i, 0)),
    )(node_pair, edge_pair, vm, W1, b1, W2, b2, W3, b3)
    return out


def _head_body(np_ref, ep_ref, vm_ref, w1_ref, b1_ref, w2_ref, b2_ref,
               w3_ref, b3_ref, out_ref):
    h_pair = jnp.concatenate([np_ref[...], ep_ref[...]], axis=-1)
    x1 = jax.nn.relu(h_pair @ w1_ref[...] + b1_ref[...])
    x2 = jax.nn.relu(x1 @ w2_ref[...] + b2_ref[...])
    out_ref[...] = (x2 @ w3_ref[...] + b3_ref[...]) * vm_ref[...]
